# Initial kernel scaffold; baseline (speedup 1.0000x reference)
import jax
import jax.numpy as jnp
from jax.experimental import pallas as pl


def kernel(input_ids, embedding_table):
    raise NotImplementedError("write your pallas kernel here")


# speedup vs baseline: 1.5678x; 1.5678x over previous
import functools

import jax
import jax.numpy as jnp
from jax import lax
from jax.experimental import pallas as pl
from jax.experimental.pallas import tpu as pltpu
from jax.experimental.pallas import tpu_sc as plsc


@functools.cache
def _build(B, V, D, NC, NS):
    NW = NC * NS
    b_per_w = B // NW
    K = 64
    n_chunks = b_per_w // K

    mesh = plsc.VectorSubcoreMesh(core_axis_name="c", subcore_axis_name="s")

    @functools.partial(
        pl.kernel,
        mesh=mesh,
        out_type=jax.ShapeDtypeStruct((B, D), jnp.float32),
        scratch_types=[
            pltpu.VMEM((n_chunks, K), jnp.int32),
            pltpu.VMEM((K, D), jnp.float32),
            pltpu.SemaphoreType.DMA,
        ],
    )
    def emb(idx_hbm, table_hbm, out_hbm, idx_v, rows_v, sem):
        wid = lax.axis_index("s") * NC + lax.axis_index("c")
        base = wid * b_per_w
        pltpu.sync_copy(idx_hbm.at[wid], idx_v)

        def body(g, _):
            pltpu.async_copy(table_hbm.at[idx_v.at[g]], rows_v, sem).wait()
            pltpu.sync_copy(rows_v, out_hbm.at[pl.ds(base + g * K, K)])
            return 0

        lax.fori_loop(0, n_chunks, body, 0)

    return emb


def kernel(input_ids, embedding_table):
    B = input_ids.size
    V, D = embedding_table.shape
    info = plsc.get_sparse_core_info()
    NC, NS = info.num_cores, info.num_subcores
    NW = NC * NS
    b_per_w = B // NW
    K = 64
    idx3 = input_ids.reshape(NW, b_per_w // K, K).astype(jnp.int32)
    out = _build(B, V, D, NC, NS)(idx3, embedding_table)
    return out.reshape(*input_ids.shape, D)


# speedup vs baseline: 1.6233x; 1.0354x over previous
import functools

import jax
import jax.numpy as jnp
from jax import lax
from jax.experimental import pallas as pl
from jax.experimental.pallas import tpu as pltpu
from jax.experimental.pallas import tpu_sc as plsc

_K = 32
_NBUF = 2


@functools.cache
def _build(B, V, D, NC, NS):
    NW = NC * NS
    b_per_w = B // NW
    n_chunks = b_per_w // _K

    mesh = plsc.VectorSubcoreMesh(core_axis_name="c", subcore_axis_name="s")

    @functools.partial(
        pl.kernel,
        mesh=mesh,
        out_type=jax.ShapeDtypeStruct((B, D), jnp.float32),
        scratch_types=[
            pltpu.VMEM((n_chunks, _K), jnp.int32),
            pltpu.VMEM((_NBUF, _K, D), jnp.float32),
            pltpu.SemaphoreType.DMA((_NBUF,)),
            pltpu.SemaphoreType.DMA((_NBUF,)),
        ],
    )
    def emb(idx_hbm, table_hbm, out_hbm, idx_v, rows_v, sem_in, sem_out):
        wid = lax.axis_index("s") * NC + lax.axis_index("c")
        base = wid * b_per_w
        pltpu.sync_copy(idx_hbm.at[wid], idx_v)

        def gather(g, b):
            return pltpu.async_copy(
                table_hbm.at[idx_v.at[g]], rows_v.at[b], sem_in.at[b])

        def put(g, b):
            return pltpu.async_copy(
                rows_v.at[b], out_hbm.at[pl.ds(base + g * _K, _K)],
                sem_out.at[b])

        inflight = {}
        for b in range(min(_NBUF, n_chunks)):
            inflight[b] = gather(b, b)
        for g in range(n_chunks):
            b = g % _NBUF
            inflight[b].wait()
            h = put(g, b)
            h.wait()
            nxt = g + _NBUF
            if nxt < n_chunks:
                inflight[b] = gather(nxt, b)

    return emb


def kernel(input_ids, embedding_table):
    B = input_ids.size
    V, D = embedding_table.shape
    info = plsc.get_sparse_core_info()
    NC, NS = info.num_cores, info.num_subcores
    NW = NC * NS
    b_per_w = B // NW
    idx3 = input_ids.reshape(NW, b_per_w // _K, _K).astype(jnp.int32)
    out = _build(B, V, D, NC, NS)(idx3, embedding_table)
    return out.reshape(*input_ids.shape, D)


# speedup vs baseline: 1.6611x; 1.0233x over previous
import functools

import jax
import jax.numpy as jnp
from jax import lax
from jax.experimental import pallas as pl
from jax.experimental.pallas import tpu as pltpu
from jax.experimental.pallas import tpu_sc as plsc

_K = 32
_NBUF = 2


@functools.cache
def _build(R, C, V, D, NC, NS):
    NW = NC * NS
    B = R * C
    b_per_w = B // NW
    n_chunks = b_per_w // _K
    n_groups = n_chunks // _NBUF
    w_per_r = C // b_per_w

    mesh = plsc.VectorSubcoreMesh(core_axis_name="c", subcore_axis_name="s")

    @functools.partial(
        pl.kernel,
        mesh=mesh,
        out_type=jax.ShapeDtypeStruct((R, C, D), jnp.float32),
        scratch_types=[
            pltpu.VMEM((n_chunks, _K), jnp.int32),
            pltpu.VMEM((_NBUF, _K, D), jnp.float32),
            pltpu.SemaphoreType.DMA((_NBUF,)),
            pltpu.SemaphoreType.DMA((_NBUF,)),
        ],
    )
    def emb(idx_hbm, table_hbm, out_hbm, idx_v, rows_v, sem_in, sem_out):
        wid = lax.axis_index("s") * NC + lax.axis_index("c")
        row = wid // w_per_r
        col = (wid % w_per_r) * b_per_w
        pltpu.sync_copy(idx_hbm.at[wid], idx_v)

        def gather(g, b):
            return pltpu.async_copy(
                table_hbm.at[idx_v.at[g]], rows_v.at[b], sem_in.at[b])

        def put(g, b):
            return pltpu.async_copy(
                rows_v.at[b], out_hbm.at[row, pl.ds(col + g * _K, _K)],
                sem_out.at[b])

        def step(g, b, refill):
            pltpu.make_async_copy(
                table_hbm.at[idx_v.at[g]], rows_v.at[b], sem_in.at[b]).wait()
            put(g, b).wait()
            if refill:
                gather(g + _NBUF, b)

        for b in range(_NBUF):
            gather(b, b)

        def group(j, _):
            for b in range(_NBUF):
                step(j * _NBUF + b, b, refill=True)
            return 0

        lax.fori_loop(0, n_groups - 1, group, 0)
        for b in range(_NBUF):
            step((n_groups - 1) * _NBUF + b, b, refill=False)

    return emb


def kernel(input_ids, embedding_table):
    R, C = input_ids.shape
    V, D = embedding_table.shape
    info = plsc.get_sparse_core_info()
    NC, NS = info.num_cores, info.num_subcores
    NW = NC * NS
    b_per_w = (R * C) // NW
    idx3 = input_ids.reshape(NW, b_per_w // _K, _K).astype(jnp.int32)
    return _build(R, C, V, D, NC, NS)(idx3, embedding_table)
